# baseline (speedup 1.0000x reference)
import jax
import jax.numpy as jnp
from jax.experimental import pallas as pl
from jax.experimental.pallas import tpu as pltpu

SUBBLK = 2048
NSTREAM = 2
STEP = SUBBLK * NSTREAM


def _top2_softmax(x, wt, bias):
    logits = jax.lax.dot_general(
        x, wt, (((1,), (0,)), ((), ())),
        preferred_element_type=jnp.float32)
    logits = logits + bias
    n_exp = logits.shape[-1]
    eidx = jax.lax.broadcasted_iota(jnp.int32, logits.shape, 1)
    m1 = jnp.max(logits, axis=-1, keepdims=True)
    i1 = jnp.min(jnp.where(logits == m1, eidx, n_exp), axis=-1, keepdims=True)
    masked = jnp.where(eidx == i1, -jnp.inf, logits)
    m2 = jnp.max(masked, axis=-1, keepdims=True)
    i2 = jnp.min(jnp.where(masked == m2, eidx, n_exp), axis=-1, keepdims=True)
    e2 = jnp.exp(m2 - m1)
    denom = 1.0 + e2
    scores = jnp.concatenate([1.0 / denom, e2 / denom], axis=1)
    idx = jnp.concatenate([i1, i2], axis=1)
    return scores, idx


def _router_block(*refs):
    x_refs = refs[:NSTREAM]
    wt_ref, b_ref = refs[NSTREAM], refs[NSTREAM + 1]
    scores_ref, idx_ref = refs[NSTREAM + 2], refs[NSTREAM + 3]
    wt = wt_ref[...]
    bias = b_ref[...]
    for k, x_ref in enumerate(x_refs):
        scores, idx = _top2_softmax(x_ref[...], wt, bias)
        scores_ref[k * SUBBLK:(k + 1) * SUBBLK, :] = scores
        idx_ref[k * SUBBLK:(k + 1) * SUBBLK, :] = idx


def kernel(x, W, b):
    n_tok, d = x.shape
    n_exp = W.shape[0]
    wt = W.T
    b2 = b.reshape(1, n_exp)
    grid = (n_tok // STEP,)
    x_specs = [
        pl.BlockSpec((SUBBLK, d), lambda i, k=k: (i * NSTREAM + k, 0))
        for k in range(NSTREAM)
    ]
    scores, idx = pl.pallas_call(
        _router_block,
        grid=grid,
        in_specs=x_specs + [
            pl.BlockSpec((d, n_exp), lambda i: (0, 0)),
            pl.BlockSpec((1, n_exp), lambda i: (0, 0)),
        ],
        out_specs=[
            pl.BlockSpec((STEP, 2), lambda i: (i, 0)),
            pl.BlockSpec((STEP, 2), lambda i: (i, 0)),
        ],
        out_shape=[
            jax.ShapeDtypeStruct((n_tok, 2), jnp.float32),
            jax.ShapeDtypeStruct((n_tok, 2), jnp.int32),
        ],
        compiler_params=pltpu.CompilerParams(
            dimension_semantics=("parallel",)),
    )(*([x] * NSTREAM), wt, b2)
    return (scores, idx)


# speedup vs baseline: 1.0240x; 1.0240x over previous
import jax
import jax.numpy as jnp
from jax.experimental import pallas as pl
from jax.experimental.pallas import tpu as pltpu

BLK = 4096


def _router_block(x_ref, wt_ref, b_ref, scores_ref, idx_ref):
    x = x_ref[...]
    logits = jax.lax.dot_general(
        x, wt_ref[...], (((1,), (0,)), ((), ())),
        preferred_element_type=jnp.float32)
    logits = logits + b_ref[...]
    n_exp = logits.shape[-1]
    eidx = jax.lax.broadcasted_iota(jnp.int32, logits.shape, 1)
    m1 = jnp.max(logits, axis=-1, keepdims=True)
    i1 = jnp.min(jnp.where(logits == m1, eidx, n_exp), axis=-1, keepdims=True)
    masked = jnp.where(eidx == i1, -jnp.inf, logits)
    m2 = jnp.max(masked, axis=-1, keepdims=True)
    i2 = jnp.min(jnp.where(masked == m2, eidx, n_exp), axis=-1, keepdims=True)
    e2 = jnp.exp(m2 - m1)
    denom = 1.0 + e2
    scores_ref[...] = jnp.concatenate([1.0 / denom, e2 / denom], axis=1)
    idx_ref[...] = jnp.concatenate([i1, i2], axis=1)


def kernel(x, W, b):
    n_tok, d = x.shape
    n_exp = W.shape[0]
    wt = W.T
    b2 = b.reshape(1, n_exp)
    grid = (n_tok // BLK,)
    scores, idx = pl.pallas_call(
        _router_block,
        grid=grid,
        in_specs=[
            pl.BlockSpec((BLK, d), lambda i: (i, 0)),
            pl.BlockSpec((d, n_exp), lambda i: (0, 0)),
            pl.BlockSpec((1, n_exp), lambda i: (0, 0)),
        ],
        out_specs=[
            pl.BlockSpec((BLK, 2), lambda i: (i, 0)),
            pl.BlockSpec((BLK, 2), lambda i: (i, 0)),
        ],
        out_shape=[
            jax.ShapeDtypeStruct((n_tok, 2), jnp.float32),
            jax.ShapeDtypeStruct((n_tok, 2), jnp.int32),
        ],
        compiler_params=pltpu.CompilerParams(
            dimension_semantics=("parallel",)),
    )(x, wt, b2)
    return (scores, idx)
